# Initial kernel scaffold; baseline (speedup 1.0000x reference)
import functools

import jax
import jax.numpy as jnp
from jax import lax
from jax.experimental import pallas as pl
from jax.experimental.pallas import tpu as pltpu
from jax.experimental.pallas import tpu_sc as plsc

DIM = 64
B_TOTAL = 16384 * 50
IDX_MINOR = 128
NC, NS = 2, 16
NW = NC * NS
B_PER_W = B_TOTAL // NW
CHUNK = 512
ROWS_PER_CHUNK = CHUNK // IDX_MINOR
STEPS = B_PER_W // CHUNK


def _sc_body(idx_hbm, table_hbm, out_hbm, idx_v, rows_v, sem):
    wid = lax.axis_index("s") * NC + lax.axis_index("c")
    idx_row0 = wid * (B_PER_W // IDX_MINOR)
    out_row0 = wid * B_PER_W

    def step(g, carry):
        pltpu.sync_copy(idx_hbm.at[pl.ds(idx_row0 + g * ROWS_PER_CHUNK,
                                         ROWS_PER_CHUNK)], idx_v)
        copies = [
            pltpu.async_copy(table_hbm.at[idx_v.at[j]],
                             rows_v.at[pl.ds(j * IDX_MINOR, IDX_MINOR)], sem)
            for j in range(ROWS_PER_CHUNK)
        ]
        for c in copies:
            c.wait()
        pltpu.sync_copy(rows_v, out_hbm.at[pl.ds(out_row0 + g * CHUNK, CHUNK)])
        return carry

    lax.fori_loop(0, STEPS, step, 0)


@jax.jit
def _embed(discard_data, table):
    idx2d = discard_data.reshape(B_TOTAL // IDX_MINOR, IDX_MINOR)
    mesh = plsc.VectorSubcoreMesh(core_axis_name="c", subcore_axis_name="s")
    out = pl.kernel(
        _sc_body,
        out_type=jax.ShapeDtypeStruct((B_TOTAL, DIM), jnp.float32),
        mesh=mesh,
        scratch_types=[
            pltpu.VMEM((ROWS_PER_CHUNK, IDX_MINOR), jnp.int32),
            pltpu.VMEM((CHUNK, DIM), jnp.float32),
            pltpu.SemaphoreType.DMA,
        ],
    )(idx2d, table)
    return out.reshape(discard_data.shape[0], discard_data.shape[1], DIM)


def kernel(discard_data, table):
    return _embed(discard_data, table)


# speedup vs baseline: 6.4042x; 6.4042x over previous
import jax
import jax.numpy as jnp
from jax import lax
from jax.experimental import pallas as pl
from jax.experimental.pallas import tpu as pltpu
from jax.experimental.pallas import tpu_sc as plsc

DIM = 64
B_TOTAL = 16384 * 50
P_TOTAL = B_TOTAL // 2
IDX_MINOR = 128
NC, NS = 2, 16
NW = NC * NS
B_PER_W = B_TOTAL // NW
CHUNK = 512
PAIRS_PER_CHUNK = CHUNK // 2
IDX_ROWS_PER_CHUNK = CHUNK // IDX_MINOR
STEPS = B_PER_W // CHUNK


def _sc_body(idx_hbm, pt_hbm, out_hbm, pt_v, idx_v, pidx_v, rows_v, sem):
    wid = lax.axis_index("s") * NC + lax.axis_index("c")
    idx_row0 = wid * (B_PER_W // IDX_MINOR)
    out_row0 = wid * (B_PER_W // 2)

    @pl.when(lax.axis_index("s") == 0)
    def _():
        pltpu.sync_copy(pt_hbm, pt_v)

    plsc.subcore_barrier()

    iota = lax.iota(jnp.int32, 16)

    def step(g, carry):
        pltpu.sync_copy(
            idx_hbm.at[pl.ds(idx_row0 + g * IDX_ROWS_PER_CHUNK,
                             IDX_ROWS_PER_CHUNK)], idx_v)
        for v in range(PAIRS_PER_CHUNK // 16):
            f0 = 32 * v
            r0 = jnp.full((16,), f0 // IDX_MINOR, jnp.int32)
            cb = f0 % IDX_MINOR
            ev = plsc.load_gather(idx_v, [r0, cb + 2 * iota])
            od = plsc.load_gather(idx_v, [r0, cb + 1 + 2 * iota])
            pidx_v[v // 8, pl.ds((v % 8) * 16, 16)] = 3 * ev + od
        copies = [
            pltpu.async_copy(pt_v.at[pidx_v.at[j]],
                             rows_v.at[pl.ds(j * IDX_MINOR, IDX_MINOR)], sem)
            for j in range(PAIRS_PER_CHUNK // IDX_MINOR)
        ]
        for c in copies:
            c.wait()
        pltpu.sync_copy(rows_v,
                        out_hbm.at[pl.ds(out_row0 + g * PAIRS_PER_CHUNK,
                                         PAIRS_PER_CHUNK)])
        return carry

    lax.fori_loop(0, STEPS, step, 0)


@jax.jit
def _embed(discard_data, table):
    idx2d = discard_data.reshape(B_TOTAL // IDX_MINOR, IDX_MINOR)
    pt = jnp.concatenate(
        [jnp.repeat(table, 3, axis=0), jnp.tile(table, (3, 1))], axis=1)
    mesh = plsc.VectorSubcoreMesh(core_axis_name="c", subcore_axis_name="s")
    out = pl.kernel(
        _sc_body,
        out_type=jax.ShapeDtypeStruct((P_TOTAL, 2 * DIM), jnp.float32),
        mesh=mesh,
        compiler_params=pltpu.CompilerParams(needs_layout_passes=False),
        scratch_types=[
            pltpu.VMEM_SHARED((9, 2 * DIM), jnp.float32),
            pltpu.VMEM((IDX_ROWS_PER_CHUNK, IDX_MINOR), jnp.int32),
            pltpu.VMEM((PAIRS_PER_CHUNK // IDX_MINOR, IDX_MINOR), jnp.int32),
            pltpu.VMEM((PAIRS_PER_CHUNK, 2 * DIM), jnp.float32),
            pltpu.SemaphoreType.DMA,
        ],
    )(idx2d, pt)
    return out.reshape(discard_data.shape[0], discard_data.shape[1], DIM)


def kernel(discard_data, table):
    return _embed(discard_data, table)


# speedup vs baseline: 6.4083x; 1.0006x over previous
import jax
import jax.numpy as jnp
from jax import lax
from jax.experimental import pallas as pl
from jax.experimental.pallas import tpu as pltpu
from jax.experimental.pallas import tpu_sc as plsc

DIM = 64
B_TOTAL = 16384 * 50
P_TOTAL = B_TOTAL // 2
IDX_MINOR = 128
NC, NS = 2, 16
NW = NC * NS
B_PER_W = B_TOTAL // NW
CHUNK = 512
PAIRS_PER_CHUNK = CHUNK // 2
IDX_ROWS_PER_CHUNK = CHUNK // IDX_MINOR
STEPS = B_PER_W // CHUNK


def _sc_body(idx_hbm, pt_hbm, out_hbm, pt_v, idx_v, pidx_v, rows_v, sem):
    wid = lax.axis_index("s") * NC + lax.axis_index("c")
    idx_row0 = wid * (B_PER_W // IDX_MINOR)
    out_row0 = wid * (B_PER_W // 2)

    @pl.when(lax.axis_index("s") == 0)
    def _():
        pltpu.sync_copy(pt_hbm, pt_v)

    plsc.subcore_barrier()

    iota = lax.iota(jnp.int32, 16)

    def step(g, carry):
        pltpu.sync_copy(
            idx_hbm.at[pl.ds(idx_row0 + g * IDX_ROWS_PER_CHUNK,
                             IDX_ROWS_PER_CHUNK)], idx_v)
        for v in range(PAIRS_PER_CHUNK // 16):
            f0 = 32 * v
            r0 = jnp.full((16,), f0 // IDX_MINOR, jnp.int32)
            cb = f0 % IDX_MINOR
            ev = plsc.load_gather(idx_v, [r0, cb + 2 * iota])
            od = plsc.load_gather(idx_v, [r0, cb + 1 + 2 * iota])
            pidx_v[v // 8, pl.ds((v % 8) * 16, 16)] = 3 * ev + od
        copies = [
            pltpu.async_copy(pt_v.at[pidx_v.at[j]],
                             rows_v.at[pl.ds(j * IDX_MINOR, IDX_MINOR)], sem)
            for j in range(PAIRS_PER_CHUNK // IDX_MINOR)
        ]
        for c in copies:
            c.wait()
        pltpu.sync_copy(rows_v,
                        out_hbm.at[pl.ds(out_row0 + g * PAIRS_PER_CHUNK,
                                         PAIRS_PER_CHUNK)])
        return carry

    lax.fori_loop(0, STEPS, step, 0)


@jax.jit
def _embed(discard_data, table):
    idx2d = discard_data.reshape(B_TOTAL // IDX_MINOR, IDX_MINOR)
    pt = jnp.concatenate(
        [jnp.repeat(table, 3, axis=0), jnp.tile(table, (3, 1))], axis=1)
    mesh = plsc.VectorSubcoreMesh(core_axis_name="c", subcore_axis_name="s")
    out = pl.kernel(
        _sc_body,
        out_type=jax.ShapeDtypeStruct((P_TOTAL, 2 * DIM), jnp.float32),
        mesh=mesh,
        compiler_params=pltpu.CompilerParams(needs_layout_passes=False,
                                             use_tc_tiling_on_sc=False),
        scratch_types=[
            pltpu.VMEM_SHARED((9, 2 * DIM), jnp.float32),
            pltpu.VMEM((IDX_ROWS_PER_CHUNK, IDX_MINOR), jnp.int32),
            pltpu.VMEM((PAIRS_PER_CHUNK // IDX_MINOR, IDX_MINOR), jnp.int32),
            pltpu.VMEM((PAIRS_PER_CHUNK, 2 * DIM), jnp.float32),
            pltpu.SemaphoreType.DMA,
        ],
    )(idx2d, pt)
    return out.reshape(discard_data.shape[0], discard_data.shape[1], DIM)


def kernel(discard_data, table):
    return _embed(discard_data, table)
